# Initial kernel scaffold; baseline (speedup 1.0000x reference)
import jax
import jax.numpy as jnp
from jax.experimental import pallas as pl


def kernel(inputs, edge_index, W_self1, W_neigh1, b1, W_self2, W_neigh2, b2):
    raise NotImplementedError("write your pallas kernel here")


# speedup vs baseline: 5.2500x; 5.2500x over previous
import functools

import jax
import jax.numpy as jnp
from jax import lax
from jax.experimental import pallas as pl
from jax.experimental.pallas import tpu as pltpu
from jax.experimental.pallas import tpu_sc as plsc

N = 10000
E = 320000
D = 128

NC = 2
NS = 16
NW = NC * NS
EPW = E // NW
CH = 80
NCHUNK = EPW // CH
NPAD = 10240
RPT = NPAD // NS
ZB = 128

_sc_mesh = plsc.VectorSubcoreMesh(
    core_axis_name="c", subcore_axis_name="s", num_cores=NC, num_subcores=NS)


def _sc_agg_body(with_deg, *refs):
    if with_deg:
        (y, src, dst, z128, z16, ones_h,
         agg_out, deg_out,
         acc, deg16, src_v, dst_v, rows_v, ones_v, zb, zb16, sem) = refs
    else:
        (y, src, dst, z128,
         agg_out,
         acc, src_v, dst_v, rows_v, zb, sem) = refs

    cid = lax.axis_index("c")
    sid = lax.axis_index("s")
    wid = cid * NS + sid
    r0 = sid * RPT

    pltpu.sync_copy(z128, zb)
    for k in range(RPT // ZB):
        pltpu.sync_copy(zb, acc.at[pl.ds(r0 + k * ZB, ZB)])
    if with_deg:
        pltpu.sync_copy(z16, zb16)
        pltpu.sync_copy(zb16, deg16.at[pl.ds(r0, RPT)])
        pltpu.sync_copy(ones_h, ones_v)
    plsc.subcore_barrier()

    ebase = wid * EPW

    def step(i, carry):
        b = ebase + i * CH
        pltpu.sync_copy(src.at[pl.ds(b, CH)], src_v)
        pltpu.sync_copy(dst.at[pl.ds(b, CH)], dst_v)
        pltpu.async_copy(y.at[src_v], rows_v, sem).wait()
        pltpu.sync_copy(rows_v, acc.at[dst_v], add=True)
        if with_deg:
            pltpu.sync_copy(ones_v, deg16.at[dst_v], add=True)
        return carry

    lax.fori_loop(0, NCHUNK, step, 0)
    plsc.subcore_barrier()

    for k in range(RPT // ZB):
        pltpu.sync_copy(acc.at[pl.ds(r0 + k * ZB, ZB)], zb)
        pltpu.sync_copy(zb, agg_out.at[cid, pl.ds(r0 + k * ZB, ZB)])
    if with_deg:
        pltpu.sync_copy(deg16.at[pl.ds(r0, RPT)], zb16)
        pltpu.sync_copy(zb16, deg_out.at[cid, pl.ds(r0, RPT)])


def _make_sc_agg(with_deg):
    out_type = [jax.ShapeDtypeStruct((NC, NPAD, D), jnp.float32)]
    scratch = [
        pltpu.VMEM_SHARED((NPAD, D), jnp.float32),
    ]
    if with_deg:
        out_type.append(jax.ShapeDtypeStruct((NC, NPAD, 16), jnp.float32))
        scratch.append(pltpu.VMEM_SHARED((NPAD, 16), jnp.float32))
    scratch += [
        pltpu.VMEM((CH,), jnp.int32),
        pltpu.VMEM((CH,), jnp.int32),
        pltpu.VMEM((CH, D), jnp.float32),
    ]
    if with_deg:
        scratch.append(pltpu.VMEM((CH, 16), jnp.float32))
    scratch.append(pltpu.VMEM((ZB, D), jnp.float32))
    if with_deg:
        scratch.append(pltpu.VMEM((RPT, 16), jnp.float32))
    scratch.append(pltpu.SemaphoreType.DMA)
    return pl.kernel(
        functools.partial(_sc_agg_body, with_deg),
        out_type=tuple(out_type),
        mesh=_sc_mesh,
        scratch_types=scratch,
        compiler_params=pltpu.CompilerParams(use_tc_tiling_on_sc=False),
    )


BN = 1000


def _tc_pre_body(x, w, o):
    o[...] = jnp.dot(x[...], w[...], preferred_element_type=jnp.float32)


def _tc_pre(x, w):
    return pl.pallas_call(
        _tc_pre_body,
        grid=(N // BN,),
        in_specs=[
            pl.BlockSpec((BN, D), lambda i: (i, 0)),
            pl.BlockSpec((D, D), lambda i: (0, 0)),
        ],
        out_specs=pl.BlockSpec((BN, D), lambda i: (i, 0)),
        out_shape=jax.ShapeDtypeStruct((N, D), jnp.float32),
    )(x, w)


def _tc_combine_body(two_outputs, *refs):
    if two_outputs:
        x, a0, a1, deg, ws, b, wn2, h, y2 = refs
    else:
        x, a0, a1, deg, ws, b, h = refs
    inv = 1.0 / jnp.maximum(deg[...], 1.0)
    hn = (a0[...] + a1[...]) * inv
    hv = jnp.maximum(
        jnp.dot(x[...], ws[...], preferred_element_type=jnp.float32)
        + hn + b[...], 0.0)
    h[...] = hv
    if two_outputs:
        y2[...] = jnp.dot(hv, wn2[...], preferred_element_type=jnp.float32)


def _tc_combine(x, a0, a1, deg, ws, b, wn2=None):
    two = wn2 is not None
    blk = pl.BlockSpec((BN, D), lambda i: (i, 0))
    full = pl.BlockSpec((D, D), lambda i: (0, 0))
    in_specs = [blk, blk, blk,
                pl.BlockSpec((BN, 1), lambda i: (i, 0)),
                full,
                pl.BlockSpec((1, D), lambda i: (0, 0))]
    args = [x, a0, a1, deg, ws, b]
    if two:
        in_specs.append(full)
        args.append(wn2)
        out_specs = (blk, blk)
        out_shape = (jax.ShapeDtypeStruct((N, D), jnp.float32),
                     jax.ShapeDtypeStruct((N, D), jnp.float32))
    else:
        out_specs = blk
        out_shape = jax.ShapeDtypeStruct((N, D), jnp.float32)
    return pl.pallas_call(
        functools.partial(_tc_combine_body, two),
        grid=(N // BN,),
        in_specs=in_specs,
        out_specs=out_specs,
        out_shape=out_shape,
    )(*args)


def kernel(inputs, edge_index, W_self1, W_neigh1, b1, W_self2, W_neigh2, b2):
    x = inputs.astype(jnp.float32)
    src = edge_index[0].astype(jnp.int32)
    dst = edge_index[1].astype(jnp.int32)
    z128 = jnp.zeros((ZB, D), jnp.float32)
    z16 = jnp.zeros((RPT, 16), jnp.float32)
    ones_h = jnp.ones((CH, 16), jnp.float32)
    b1r = b1.reshape(1, D).astype(jnp.float32)
    b2r = b2.reshape(1, D).astype(jnp.float32)

    sc_agg_deg = _make_sc_agg(True)
    sc_agg = _make_sc_agg(False)

    y1 = _tc_pre(x, W_neigh1)
    agg1, degp = sc_agg_deg(y1, src, dst, z128, z16, ones_h)
    deg = (degp[0, :N, 0] + degp[1, :N, 0]).reshape(N, 1)
    h, y2 = _tc_combine(x, agg1[0, :N], agg1[1, :N], deg, W_self1, b1r,
                        W_neigh2)

    (agg2,) = sc_agg(y2, src, dst, z128)
    out = _tc_combine(h, agg2[0, :N], agg2[1, :N], deg, W_self2, b2r)
    return out


# speedup vs baseline: 9.4360x; 1.7973x over previous
import functools

import jax
import jax.numpy as jnp
from jax import lax
from jax.experimental import pallas as pl
from jax.experimental.pallas import tpu as pltpu
from jax.experimental.pallas import tpu_sc as plsc

N = 10000
E = 320000
D = 128
DW1 = 144

NC = 2
NS = 16
NW = NC * NS
EPW = E // NW
CH = 40
NCHUNK = EPW // CH
SCH = 5
NSUP = NCHUNK // SCH
NPAD = 10240
RPT = NPAD // NS

_sc_mesh = plsc.VectorSubcoreMesh(
    core_axis_name="c", subcore_axis_name="s", num_cores=NC, num_subcores=NS)


def _sc_agg_body(dw, *refs):
    (y, src, dst, zfill, agg_out,
     acc, src_v, dst_v, rows_v, *sems) = refs
    gsems = sems[:SCH]
    ssems = sems[SCH:2 * SCH]
    isem0, isem1 = sems[2 * SCH], sems[2 * SCH + 1]

    cid = lax.axis_index("c")
    sid = lax.axis_index("s")
    wid = cid * NS + sid
    r0 = sid * RPT

    zb = rows_v.at[0]
    pltpu.sync_copy(zfill, zb)
    for k in range(RPT // CH):
        pltpu.sync_copy(zb, acc.at[pl.ds(r0 + k * CH, CH)])
    for j in range(SCH):
        pltpu.async_copy(zfill, rows_v.at[j], ssems[j])
    plsc.subcore_barrier()

    crow0 = wid * NCHUNK

    def idx_start(sup_i, p, isem):
        cr = crow0 + sup_i * SCH
        pltpu.async_copy(src.at[pl.ds(cr, SCH)], src_v.at[p], isem)
        pltpu.async_copy(dst.at[pl.ds(cr, SCH)], dst_v.at[p], isem)

    def idx_wait(sup_i, p, isem):
        cr = crow0 + sup_i * SCH
        pltpu.make_async_copy(src.at[pl.ds(cr, SCH)], src_v.at[p], isem).wait()
        pltpu.make_async_copy(dst.at[pl.ds(cr, SCH)], dst_v.at[p], isem).wait()

    def slot_wait(j):
        pltpu.make_async_copy(zfill, rows_v.at[j], ssems[j]).wait()

    def proc(sup_i, p, isem):
        idx_wait(sup_i, p, isem)
        gathers = []
        for j in range(SCH):
            slot_wait(j)
            gathers.append(pltpu.async_copy(
                y.at[src_v.at[p, j]], rows_v.at[j], gsems[j]))
        for j in range(SCH):
            gathers[j].wait()
            pltpu.async_copy(
                rows_v.at[j], acc.at[dst_v.at[p, j]], ssems[j], add=True)
        idx_start(sup_i + 2, p, isem)

    idx_start(0, 0, isem0)
    idx_start(1, 1, isem1)

    def step(t, carry):
        proc(2 * t, 0, isem0)
        proc(2 * t + 1, 1, isem1)
        return carry

    lax.fori_loop(0, NSUP // 2, step, 0)
    for j in range(SCH):
        slot_wait(j)
    idx_wait(NSUP, 0, isem0)
    idx_wait(NSUP + 1, 1, isem1)
    plsc.subcore_barrier()

    zb2 = rows_v.at[0]
    for k in range(RPT // CH):
        pltpu.sync_copy(acc.at[pl.ds(r0 + k * CH, CH)], zb2)
        pltpu.sync_copy(zb2, agg_out.at[cid, pl.ds(r0 + k * CH, CH)])


def _make_sc_agg(dw):
    return pl.kernel(
        functools.partial(_sc_agg_body, dw),
        out_type=(jax.ShapeDtypeStruct((NC, NPAD, dw), jnp.float32),),
        mesh=_sc_mesh,
        scratch_types=[
            pltpu.VMEM_SHARED((NPAD, dw), jnp.float32),
            pltpu.VMEM((2, SCH, CH), jnp.int32),
            pltpu.VMEM((2, SCH, CH), jnp.int32),
            pltpu.VMEM((SCH, CH, dw), jnp.float32),
        ] + [pltpu.SemaphoreType.DMA] * (2 * SCH + 2),
        compiler_params=pltpu.CompilerParams(use_tc_tiling_on_sc=False),
    )


BN = 1000


def _tc_pre_body(x, w, o):
    mm = jnp.dot(x[...], w[...], preferred_element_type=jnp.float32)
    o[...] = jnp.concatenate(
        [mm, jnp.ones((BN, DW1 - D), jnp.float32)], axis=1)


def _tc_pre(x, w):
    return pl.pallas_call(
        _tc_pre_body,
        grid=(N // BN,),
        in_specs=[
            pl.BlockSpec((BN, D), lambda i: (i, 0)),
            pl.BlockSpec((D, D), lambda i: (0, 0)),
        ],
        out_specs=pl.BlockSpec((BN, DW1), lambda i: (i, 0)),
        out_shape=jax.ShapeDtypeStruct((N, DW1), jnp.float32),
    )(x, w)


def _tc_combine1_body(x, a0, a1, ws, b, wn2, h, y2, dego):
    aa = a0[...] + a1[...]
    degv = aa[:, D:D + 1]
    inv = 1.0 / jnp.maximum(degv, 1.0)
    hv = jnp.maximum(
        jnp.dot(x[...], ws[...], preferred_element_type=jnp.float32)
        + aa[:, :D] * inv + b[...], 0.0)
    h[...] = hv
    y2[...] = jnp.dot(hv, wn2[...], preferred_element_type=jnp.float32)
    dego[...] = degv


def _tc_combine1(x, a0, a1, ws, b, wn2):
    blk = pl.BlockSpec((BN, D), lambda i: (i, 0))
    wblk = pl.BlockSpec((BN, DW1), lambda i: (i, 0))
    full = pl.BlockSpec((D, D), lambda i: (0, 0))
    return pl.pallas_call(
        _tc_combine1_body,
        grid=(N // BN,),
        in_specs=[blk, wblk, wblk, full,
                  pl.BlockSpec((1, D), lambda i: (0, 0)), full],
        out_specs=(blk, blk, pl.BlockSpec((BN, 1), lambda i: (i, 0))),
        out_shape=(jax.ShapeDtypeStruct((N, D), jnp.float32),
                   jax.ShapeDtypeStruct((N, D), jnp.float32),
                   jax.ShapeDtypeStruct((N, 1), jnp.float32)),
    )(x, a0, a1, ws, b, wn2)


def _tc_combine2_body(x, a0, a1, deg, ws, b, o):
    inv = 1.0 / jnp.maximum(deg[...], 1.0)
    o[...] = jnp.maximum(
        jnp.dot(x[...], ws[...], preferred_element_type=jnp.float32)
        + (a0[...] + a1[...]) * inv + b[...], 0.0)


def _tc_combine2(x, a0, a1, deg, ws, b):
    blk = pl.BlockSpec((BN, D), lambda i: (i, 0))
    full = pl.BlockSpec((D, D), lambda i: (0, 0))
    return pl.pallas_call(
        _tc_combine2_body,
        grid=(N // BN,),
        in_specs=[blk, blk, blk,
                  pl.BlockSpec((BN, 1), lambda i: (i, 0)),
                  full,
                  pl.BlockSpec((1, D), lambda i: (0, 0))],
        out_specs=blk,
        out_shape=jax.ShapeDtypeStruct((N, D), jnp.float32),
    )(x, a0, a1, deg, ws, b)


def kernel(inputs, edge_index, W_self1, W_neigh1, b1, W_self2, W_neigh2, b2):
    x = inputs.astype(jnp.float32)
    src = jnp.pad(edge_index[0].astype(jnp.int32).reshape(E // CH, CH),
                  ((0, 2 * SCH), (0, 0)))
    dst = jnp.pad(edge_index[1].astype(jnp.int32).reshape(E // CH, CH),
                  ((0, 2 * SCH), (0, 0)))
    zfill1 = jnp.zeros((CH, DW1), jnp.float32)
    zfill2 = jnp.zeros((CH, D), jnp.float32)
    b1r = b1.reshape(1, D).astype(jnp.float32)
    b2r = b2.reshape(1, D).astype(jnp.float32)

    sc_agg1 = _make_sc_agg(DW1)
    sc_agg2 = _make_sc_agg(D)

    y1 = _tc_pre(x, W_neigh1)
    (agg1,) = sc_agg1(y1, src, dst, zfill1)
    h, y2, deg = _tc_combine1(x, agg1[0, :N], agg1[1, :N], W_self1, b1r,
                              W_neigh2)

    (agg2,) = sc_agg2(y2, src, dst, zfill2)
    out = _tc_combine2(h, agg2[0, :N], agg2[1, :N], deg, W_self2, b2r)
    return out
